# baseline (speedup 1.0000x reference)
import functools

import jax
import jax.numpy as jnp
from jax import lax
from jax.experimental import pallas as pl
from jax.experimental.pallas import tpu as pltpu

B = 1024
S = 51
IN = 512
N = B * S
E = 200000

_T1 = 512
_T2 = 2000
_T3B = 8


def _lrelu(v):
    return jnp.where(v >= 0, v, 0.01 * v)


def _tc1_body(xb, pb, m2w1, m2b1, m2w2, m2b2, m2w3, m2b3,
              m3w1, m3b1, m3w2, m3b2, m3w3, m3b3,
              encw, encb, ew1, wn, ws,
              u_o, un_o, zn_o, zs_o):
    f32 = jnp.float32
    h = jax.nn.relu(jnp.dot(xb[...], m2w1[...], preferred_element_type=f32) + m2b1[...])
    h = jax.nn.relu(jnp.dot(h, m2w2[...], preferred_element_type=f32) + m2b2[...])
    h = jax.nn.relu(jnp.dot(h, m2w3[...], preferred_element_type=f32) + m2b3[...])
    p = jax.nn.relu(jnp.dot(pb[...], m3w1[...], preferred_element_type=f32) + m3b1[...])
    p = jax.nn.relu(jnp.dot(p, m3w2[...], preferred_element_type=f32) + m3b2[...])
    p = jax.nn.relu(jnp.dot(p, m3w3[...], preferred_element_type=f32) + m3b3[...])
    ew = encw[...]
    z = jax.nn.relu(jnp.dot(h, ew[0:64, :], preferred_element_type=f32)
                    + jnp.dot(p, ew[64:128, :], preferred_element_type=f32)
                    + encb[...])
    u = jnp.dot(z, ew1[...], preferred_element_type=f32)
    u_o[...] = u
    un_o[...] = -u
    zn_o[...] = jnp.dot(z, wn[...], preferred_element_type=f32)
    zs_o[...] = jnp.dot(z, ws[...], preferred_element_type=f32)


def _tc1(x2d, p2d, prm):
    grid = (N // _T1,)
    row = lambda i: (i, 0)
    full = lambda i: (0, 0)

    def wspec(a):
        return pl.BlockSpec(a.shape, full)

    weights = [prm['m2_w1'], prm['m2_b1'], prm['m2_w2'], prm['m2_b2'],
               prm['m2_w3'], prm['m2_b3'],
               prm['m3_w1'], prm['m3_b1'], prm['m3_w2'], prm['m3_b2'],
               prm['m3_w3'], prm['m3_b3'],
               prm['enc_w'], prm['enc_b'], prm['e_w1'],
               prm['sage_wn'], prm['sage_ws']]
    return pl.pallas_call(
        _tc1_body,
        grid=grid,
        in_specs=[pl.BlockSpec((_T1, IN), row), pl.BlockSpec((_T1, 7), row)]
                 + [wspec(w) for w in weights],
        out_specs=[pl.BlockSpec((_T1, 256), row), pl.BlockSpec((_T1, 256), row),
                   pl.BlockSpec((_T1, 128), row), pl.BlockSpec((_T1, 128), row)],
        out_shape=[jax.ShapeDtypeStruct((N, 256), jnp.float32),
                   jax.ShapeDtypeStruct((N, 256), jnp.float32),
                   jax.ShapeDtypeStruct((N, 128), jnp.float32),
                   jax.ShapeDtypeStruct((N, 128), jnp.float32)],
    )(x2d, p2d, *weights)


def _tc2_body(db, b1, w2, b2, w3, b3, w4, b4, e_o):
    f32 = jnp.float32
    t = _lrelu(db[...] + b1[...])
    t = _lrelu(jnp.dot(t, w2[...], preferred_element_type=f32) + b2[...])
    t = _lrelu(jnp.dot(t, w3[...], preferred_element_type=f32) + b3[...])
    logit = jnp.dot(t, w4[...], preferred_element_type=f32) + b4[...]
    e_o[...] = jax.nn.sigmoid(logit)


def _tc2(diff, prm):
    grid = (E // _T2,)
    row = lambda i: (i, 0)
    full = lambda i: (0, 0)
    weights = [prm['e_b1'], prm['e_w2'], prm['e_b2'],
               prm['e_w3'], prm['e_b3'], prm['e_w4'], prm['e_b4']]
    return pl.pallas_call(
        _tc2_body,
        grid=grid,
        in_specs=[pl.BlockSpec((_T2, 256), row)]
                 + [pl.BlockSpec(w.shape, full) for w in weights],
        out_specs=pl.BlockSpec((_T2, 1), row),
        out_shape=jax.ShapeDtypeStruct((E, 1), jnp.float32),
    )(diff, *weights)


def _tc3_body(zsb, aggb, degb, sageb, decw, decb, a_o, est_o):
    f32 = jnp.float32
    rows = _T3B * S
    agg = aggb[0] + aggb[1]
    deg = degb[0, :, 0:1] + degb[1, :, 0:1]
    neigh = agg / jnp.maximum(deg, 1.0)
    A = zsb[...] + neigh + sageb[...]
    est = jnp.dot(A, decw[...], preferred_element_type=f32) + decb[...]
    gid = lax.broadcasted_iota(jnp.int32, (_T3B, rows), 0)
    rid = lax.broadcasted_iota(jnp.int32, (_T3B, rows), 1)
    sel0 = (rid == gid * S).astype(f32)
    est_o[...] = jnp.dot(sel0, est, preferred_element_type=f32)
    Al = _lrelu(A)
    grp = (rid // S == gid).astype(f32)
    ssq = jnp.dot(grp, Al * Al, preferred_element_type=f32)
    nrm = jnp.maximum(jnp.sqrt(ssq), 1e-12)
    rownrm = jnp.dot(grp.T, nrm, preferred_element_type=f32)
    a_o[...] = Al / rownrm


def _tc3(zs, aggp, degp, prm):
    rows = _T3B * S
    grid = (N // rows,)
    row = lambda i: (i, 0)
    full = lambda i: (0, 0)
    return pl.pallas_call(
        _tc3_body,
        grid=grid,
        in_specs=[pl.BlockSpec((rows, 128), row),
                  pl.BlockSpec((2, rows, 128), lambda i: (0, i, 0)),
                  pl.BlockSpec((2, rows, 16), lambda i: (0, i, 0)),
                  pl.BlockSpec(prm['sage_b'].shape, full),
                  pl.BlockSpec(prm['dec_w'].shape, full),
                  pl.BlockSpec(prm['dec_b'].shape, full)],
        out_specs=[pl.BlockSpec((rows, 128), row),
                   pl.BlockSpec((_T3B, 7), row)],
        out_shape=[jax.ShapeDtypeStruct((N, 128), jnp.float32),
                   jax.ShapeDtypeStruct((B, 7), jnp.float32)],
    )(zs, aggp, degp, prm['sage_b'], prm['dec_w'], prm['dec_b'])


def kernel(x, x_pose, edge_index, params):
    prm = dict(params)
    for k in ['m2_b1', 'm2_b2', 'm2_b3', 'm3_b1', 'm3_b2', 'm3_b3',
              'enc_b', 'e_b1', 'e_b2', 'e_b3', 'e_b4', 'sage_b', 'dec_b']:
        prm[k] = prm[k].reshape(1, -1)
    x2d = x.reshape(N, IN)
    p2d = x_pose.reshape(N, 7)
    src = edge_index[0]
    dst = edge_index[1]

    u, un, zn, zs = _tc1(x2d, p2d, prm)

    diff = u[src] + un[dst]
    e = _tc2(diff, prm)
    msg = zn[src] * e
    agg = jax.ops.segment_sum(msg, dst, num_segments=N)
    deg = jax.ops.segment_sum(jnp.ones((E,), jnp.float32), dst, num_segments=N)
    aggp = jnp.stack([agg, jnp.zeros_like(agg)])
    degp = jnp.broadcast_to(deg[None, :, None], (1, N, 16))
    degp = jnp.concatenate([degp, jnp.zeros_like(degp)], axis=0)

    a2d, est0 = _tc3(zs, aggp, degp, prm)
    A = a2d.reshape(B, S, 128)
    pos = est0[:, 0:3]
    ori = est0[:, 3:7]
    return (A, e, pos, ori)


# speedup vs baseline: 1.4437x; 1.4437x over previous
import functools

import jax
import jax.numpy as jnp
from jax import lax
from jax.experimental import pallas as pl
from jax.experimental.pallas import tpu as pltpu
from jax.experimental.pallas import tpu_sc as plsc

B = 1024
S = 51
IN = 512
N = B * S
E = 200000

_NC = 2
_NS = 16
_NW = _NC * _NS
_EPW = 6256
_EPAD = _NW * _EPW
_GC = 136

_T1 = 512
_T2 = 3128
_T3B = 8


def _lrelu(v):
    return jnp.where(v >= 0, v, 0.01 * v)


def _tc1_body(xb, pb, m2w1, m2b1, m2w2, m2b2, m2w3, m2b3,
              m3w1, m3b1, m3w2, m3b2, m3w3, m3b3,
              encw, encb, ew1, wn, ws,
              u_o, un_o, zn_o, zs_o):
    f32 = jnp.float32
    h = jax.nn.relu(jnp.dot(xb[...], m2w1[...], preferred_element_type=f32) + m2b1[...])
    h = jax.nn.relu(jnp.dot(h, m2w2[...], preferred_element_type=f32) + m2b2[...])
    h = jax.nn.relu(jnp.dot(h, m2w3[...], preferred_element_type=f32) + m2b3[...])
    p = jax.nn.relu(jnp.dot(pb[...], m3w1[...], preferred_element_type=f32) + m3b1[...])
    p = jax.nn.relu(jnp.dot(p, m3w2[...], preferred_element_type=f32) + m3b2[...])
    p = jax.nn.relu(jnp.dot(p, m3w3[...], preferred_element_type=f32) + m3b3[...])
    ew = encw[...]
    z = jax.nn.relu(jnp.dot(h, ew[0:64, :], preferred_element_type=f32)
                    + jnp.dot(p, ew[64:128, :], preferred_element_type=f32)
                    + encb[...])
    u = jnp.dot(z, ew1[...], preferred_element_type=f32)
    u_o[...] = u
    un_o[...] = -u
    zn_o[...] = jnp.dot(z, wn[...], preferred_element_type=f32)
    zs_o[...] = jnp.dot(z, ws[...], preferred_element_type=f32)


def _tc1(x2d, p2d, prm):
    grid = (N // _T1,)
    row = lambda i: (i, 0)
    full = lambda i: (0, 0)

    def wspec(a):
        return pl.BlockSpec(a.shape, full)

    weights = [prm['m2_w1'], prm['m2_b1'], prm['m2_w2'], prm['m2_b2'],
               prm['m2_w3'], prm['m2_b3'],
               prm['m3_w1'], prm['m3_b1'], prm['m3_w2'], prm['m3_b2'],
               prm['m3_w3'], prm['m3_b3'],
               prm['enc_w'], prm['enc_b'], prm['e_w1'],
               prm['sage_wn'], prm['sage_ws']]
    return pl.pallas_call(
        _tc1_body,
        grid=grid,
        in_specs=[pl.BlockSpec((_T1, IN), row), pl.BlockSpec((_T1, 7), row)]
                 + [wspec(w) for w in weights],
        out_specs=[pl.BlockSpec((_T1, 256), row), pl.BlockSpec((_T1, 256), row),
                   pl.BlockSpec((_T1, 128), row), pl.BlockSpec((_T1, 128), row)],
        out_shape=[jax.ShapeDtypeStruct((N, 256), jnp.float32),
                   jax.ShapeDtypeStruct((N, 256), jnp.float32),
                   jax.ShapeDtypeStruct((N, 128), jnp.float32),
                   jax.ShapeDtypeStruct((N, 128), jnp.float32)],
    )(x2d, p2d, *weights)


def _g1_body(u_hbm, un_hbm, src_hbm, dst_hbm, diff_hbm,
             sidx, didx, rs, rd, sem1, sem2):
    wid = lax.axis_index("s") * _NC + lax.axis_index("c")
    base = wid * _EPW
    pltpu.sync_copy(src_hbm.at[pl.ds(base, _EPW)], sidx)
    pltpu.sync_copy(dst_hbm.at[pl.ds(base, _EPW)], didx)

    def chunk(k, carry):
        off = k * _GC
        cp1 = pltpu.async_copy(u_hbm.at[sidx.at[pl.ds(off, _GC)]], rs, sem1)
        cp2 = pltpu.async_copy(un_hbm.at[didx.at[pl.ds(off, _GC)]], rd, sem2)
        cp1.wait()
        cp2.wait()

        def row(r, c2):
            for j in range(16):
                sl = pl.ds(j * 16, 16)
                rs[r, sl] = rs[r, sl] + rd[r, sl]
            return c2

        lax.fori_loop(0, _GC, row, 0)
        pltpu.sync_copy(rs, diff_hbm.at[pl.ds(base + off, _GC)])
        return carry

    lax.fori_loop(0, _EPW // _GC, chunk, 0)


def _g1(u, un, src_p, dst_p):
    mesh = plsc.VectorSubcoreMesh(core_axis_name="c", subcore_axis_name="s")
    f = pl.kernel(
        _g1_body,
        out_type=jax.ShapeDtypeStruct((_EPAD, 256), jnp.float32),
        mesh=mesh,
        scratch_types=[
            pltpu.VMEM((_EPW,), jnp.int32),
            pltpu.VMEM((_EPW,), jnp.int32),
            pltpu.VMEM((_GC, 256), jnp.float32),
            pltpu.VMEM((_GC, 256), jnp.float32),
            pltpu.SemaphoreType.DMA,
            pltpu.SemaphoreType.DMA,
        ],
    )
    return f(u, un, src_p, dst_p)


def _tc2_body(db, b1, w2, b2, w3, b3, w4, b4, e_o):
    f32 = jnp.float32
    t = _lrelu(db[...] + b1[...])
    t = _lrelu(jnp.dot(t, w2[...], preferred_element_type=f32) + b2[...])
    t = _lrelu(jnp.dot(t, w3[...], preferred_element_type=f32) + b3[...])
    logit = jnp.dot(t, w4[...], preferred_element_type=f32) + b4[...]
    e_o[...] = jax.nn.sigmoid(logit)


def _tc2(diff, prm):
    grid = (_EPAD // _T2,)
    row = lambda i: (i, 0)
    full = lambda i: (0, 0)
    weights = [prm['e_b1'], prm['e_w2'], prm['e_b2'],
               prm['e_w3'], prm['e_b3'], prm['e_w4'], prm['e_b4']]
    return pl.pallas_call(
        _tc2_body,
        grid=grid,
        in_specs=[pl.BlockSpec((_T2, 256), row)]
                 + [pl.BlockSpec(w.shape, full) for w in weights],
        out_specs=pl.BlockSpec((_T2, 1), row),
        out_shape=jax.ShapeDtypeStruct((_EPAD, 1), jnp.float32),
    )(diff, *weights)


def _tc3_body(zsb, aggb, degb, sageb, decw, decb, a_o, est_o):
    f32 = jnp.float32
    rows = _T3B * S
    agg = aggb[0] + aggb[1]
    deg = degb[0, :, 0:1] + degb[1, :, 0:1]
    neigh = agg / jnp.maximum(deg, 1.0)
    A = zsb[...] + neigh + sageb[...]
    est = jnp.dot(A, decw[...], preferred_element_type=f32) + decb[...]
    gid = lax.broadcasted_iota(jnp.int32, (_T3B, rows), 0)
    rid = lax.broadcasted_iota(jnp.int32, (_T3B, rows), 1)
    sel0 = (rid == gid * S).astype(f32)
    est_o[...] = jnp.dot(sel0, est, preferred_element_type=f32)
    Al = _lrelu(A)
    grp = (rid // S == gid).astype(f32)
    ssq = jnp.dot(grp, Al * Al, preferred_element_type=f32)
    nrm = jnp.maximum(jnp.sqrt(ssq), 1e-12)
    rownrm = jnp.dot(grp.T, nrm, preferred_element_type=f32)
    a_o[...] = Al / rownrm


def _tc3(zs, aggp, degp, prm):
    rows = _T3B * S
    grid = (N // rows,)
    row = lambda i: (i, 0)
    full = lambda i: (0, 0)
    return pl.pallas_call(
        _tc3_body,
        grid=grid,
        in_specs=[pl.BlockSpec((rows, 128), row),
                  pl.BlockSpec((2, rows, 128), lambda i: (0, i, 0)),
                  pl.BlockSpec((2, rows, 16), lambda i: (0, i, 0)),
                  pl.BlockSpec(prm['sage_b'].shape, full),
                  pl.BlockSpec(prm['dec_w'].shape, full),
                  pl.BlockSpec(prm['dec_b'].shape, full)],
        out_specs=[pl.BlockSpec((rows, 128), row),
                   pl.BlockSpec((_T3B, 7), row)],
        out_shape=[jax.ShapeDtypeStruct((N, 128), jnp.float32),
                   jax.ShapeDtypeStruct((B, 7), jnp.float32)],
    )(zs, aggp, degp, prm['sage_b'], prm['dec_w'], prm['dec_b'])


def kernel(x, x_pose, edge_index, params):
    prm = dict(params)
    for k in ['m2_b1', 'm2_b2', 'm2_b3', 'm3_b1', 'm3_b2', 'm3_b3',
              'enc_b', 'e_b1', 'e_b2', 'e_b3', 'e_b4', 'sage_b', 'dec_b']:
        prm[k] = prm[k].reshape(1, -1)
    x2d = x.reshape(N, IN)
    p2d = x_pose.reshape(N, 7)
    pad = jnp.zeros((_EPAD - E,), jnp.int32)
    src_p = jnp.concatenate([edge_index[0], pad])
    dst_p = jnp.concatenate([edge_index[1], pad])

    u, un, zn, zs = _tc1(x2d, p2d, prm)

    diff = _g1(u, un, src_p, dst_p)
    e_pad = _tc2(diff, prm)
    e = e_pad[:E]

    src = edge_index[0]
    dst = edge_index[1]
    msg = zn[src] * e
    agg = jax.ops.segment_sum(msg, dst, num_segments=N)
    deg = jax.ops.segment_sum(jnp.ones((E,), jnp.float32), dst, num_segments=N)
    aggp = jnp.stack([agg, jnp.zeros_like(agg)])
    degp = jnp.broadcast_to(deg[None, :, None], (1, N, 16))
    degp = jnp.concatenate([degp, jnp.zeros_like(degp)], axis=0)

    a2d, est0 = _tc3(zs, aggp, degp, prm)
    A = a2d.reshape(B, S, 128)
    pos = est0[:, 0:3]
    ori = est0[:, 3:7]
    return (A, e, pos, ori)
